# baseline (speedup 1.0000x reference)
import numpy as np

import jax
import jax.numpy as jnp
from jax.experimental import pallas as pl
from jax.experimental.pallas import tpu as pltpu
from jax.sharding import Mesh, NamedSharding, PartitionSpec as P

N = 10000
F = 128

_VMEM_PARAMS = pltpu.CompilerParams(vmem_limit_bytes=128 * 1024 * 1024)
_CONST = lambda i: (0, 0)


def _elu(v):
    return jnp.where(v > 0, v, jnp.exp(jnp.minimum(v, 0.0)) - 1.0)


def _bf16_dot(a, b):
    return jnp.dot(
        a.astype(jnp.bfloat16),
        b.astype(jnp.bfloat16),
        preferred_element_type=jnp.float32,
    )


def _fused_kernel_1dev(a_ref, x_ref, w1_ref, b1_ref, wo_ref, bo_ref, o_ref,
                       h0_ref, h1_ref):
    i = pl.program_id(0)
    nbl = pl.num_programs(0) // 2

    @pl.when(i == 0)
    def _():
        h0_ref[:] = (_bf16_dot(x_ref[:], w1_ref[:]) + b1_ref[:]).astype(
            jnp.bfloat16
        )

    @pl.when(i < nbl)
    def _():
        acc = jnp.dot(
            a_ref[:].astype(jnp.bfloat16),
            h0_ref[:],
            preferred_element_type=jnp.float32,
        )
        bm = a_ref.shape[0]
        h1_ref[pl.ds(i * bm, bm), :] = _elu(acc).astype(jnp.bfloat16)

    @pl.when(i >= nbl)
    def _():
        acc = jnp.dot(
            a_ref[:].astype(jnp.bfloat16),
            h1_ref[:],
            preferred_element_type=jnp.float32,
        )
        t = _elu(acc)
        o_ref[:] = _bf16_dot(t, wo_ref[:]) + bo_ref[:]


def _kernel_1dev(x, adjs, W1t, b1r, Woutt, boutr):
    bm = 400
    nbl = N // bm
    return pl.pallas_call(
        _fused_kernel_1dev,
        grid=(2 * nbl,),
        in_specs=[
            pl.BlockSpec((bm, N), lambda i: (jax.lax.rem(i, nbl), 0)),
            pl.BlockSpec((N, F), _CONST),
            pl.BlockSpec((F, F), _CONST),
            pl.BlockSpec((1, F), _CONST),
            pl.BlockSpec((F, F), _CONST),
            pl.BlockSpec((1, F), _CONST),
        ],
        out_specs=pl.BlockSpec((bm, F), lambda i: (jnp.maximum(i - nbl, 0), 0)),
        out_shape=jax.ShapeDtypeStruct((N, F), jnp.float32),
        scratch_shapes=[
            pltpu.VMEM((N, F), jnp.bfloat16),
            pltpu.VMEM((N, F), jnp.bfloat16),
        ],
        compiler_params=_VMEM_PARAMS,
    )(adjs, x, W1t, b1r, Woutt, boutr)


def _layer1_kernel(a_ref, x_ref, w1_ref, b1_ref, o_ref, h0_ref):
    i = pl.program_id(0)

    @pl.when(i == 0)
    def _():
        h0_ref[:] = (_bf16_dot(x_ref[:], w1_ref[:]) + b1_ref[:]).astype(
            jnp.bfloat16
        )

    acc = jnp.dot(
        a_ref[:].astype(jnp.bfloat16),
        h0_ref[:],
        preferred_element_type=jnp.float32,
    )
    o_ref[:] = _elu(acc).astype(jnp.bfloat16)


def _layer2_kernel(a_ref, h1_ref, wo_ref, bo_ref, o_ref):
    acc = jnp.dot(
        a_ref[:].astype(jnp.bfloat16),
        h1_ref[:],
        preferred_element_type=jnp.float32,
    )
    t = _elu(acc)
    o_ref[:] = _bf16_dot(t, wo_ref[:]) + bo_ref[:]


def _make_sharded_impl(devs):
    ndev = len(devs)
    rows_loc = N // ndev
    bm = 200
    nbl = rows_loc // bm
    mesh = Mesh(np.array(devs), ("r",))
    rep = NamedSharding(mesh, P())
    row_sh = NamedSharding(mesh, P("r", None))

    def shard_fn(a_sh, x, w1t, b1r, wot, bor):
        h1_loc = pl.pallas_call(
            _layer1_kernel,
            grid=(nbl,),
            in_specs=[
                pl.BlockSpec((bm, N), lambda i: (i, 0)),
                pl.BlockSpec((N, F), _CONST),
                pl.BlockSpec((F, F), _CONST),
                pl.BlockSpec((1, F), _CONST),
            ],
            out_specs=pl.BlockSpec((bm, F), lambda i: (i, 0)),
            out_shape=jax.ShapeDtypeStruct((rows_loc, F), jnp.bfloat16),
            scratch_shapes=[pltpu.VMEM((N, F), jnp.bfloat16)],
            compiler_params=_VMEM_PARAMS,
        )(a_sh, x, w1t, b1r)
        h1 = jax.lax.all_gather(h1_loc, "r", axis=0, tiled=True)
        return pl.pallas_call(
            _layer2_kernel,
            grid=(nbl,),
            in_specs=[
                pl.BlockSpec((bm, N), lambda i: (i, 0)),
                pl.BlockSpec((N, F), _CONST),
                pl.BlockSpec((F, F), _CONST),
                pl.BlockSpec((1, F), _CONST),
            ],
            out_specs=pl.BlockSpec((bm, F), lambda i: (i, 0)),
            out_shape=jax.ShapeDtypeStruct((rows_loc, F), jnp.float32),
            compiler_params=_VMEM_PARAMS,
        )(a_sh, h1, wot, bor)

    sharded = jax.shard_map(
        shard_fn,
        mesh=mesh,
        in_specs=(P("r", None), P(), P(), P(), P(), P()),
        out_specs=P("r", None),
        check_vma=False,
    )

    def impl(x, adjs, W1, b1, Wout, bout):
        return sharded(
            adjs, x, W1.T, b1.reshape(1, F), Wout.T, bout.reshape(1, F)
        )

    return jax.jit(
        impl,
        in_shardings=(rep, row_sh, rep, rep, rep, rep),
        out_shardings=row_sh,
    )


def _make_impl():
    try:
        devs = jax.devices()
    except RuntimeError:
        devs = []
    if len(devs) >= 2 and devs[0].platform == "tpu":
        return _make_sharded_impl(devs[:2])

    @jax.jit
    def impl(x, adjs, W1, b1, Wout, bout):
        return _kernel_1dev(
            x, adjs, W1.T, b1.reshape(1, F), Wout.T, bout.reshape(1, F)
        )

    return impl


_IMPL = None


def kernel(x, adjs, W1, b1, Wout, bout):
    global _IMPL
    if _IMPL is None:
        _IMPL = _make_impl()
    return _IMPL(x, adjs, W1, b1, Wout, bout)


# speedup vs baseline: 3.3380x; 3.3380x over previous
import jax
import jax.numpy as jnp
from jax.experimental import pallas as pl
from jax.experimental.pallas import tpu as pltpu

N = 10000
F = 128
BM = 400
NB = N // BM

_CONST = lambda i: (0, 0)


def _elu(v):
    return jnp.where(v > 0, v, jnp.exp(jnp.minimum(v, 0.0)) - 1.0)


def _bf16_dot(a, b):
    return jnp.dot(
        a.astype(jnp.bfloat16),
        b.astype(jnp.bfloat16),
        preferred_element_type=jnp.float32,
    )


def _fused_kernel(a_ref, x_ref, w1_ref, b1_ref, wo_ref, bo_ref, o_ref,
                  h0_ref, h1_ref):
    i = pl.program_id(0)

    @pl.when(i == 0)
    def _():
        h0_ref[:] = (_bf16_dot(x_ref[:], w1_ref[:]) + b1_ref[:]).astype(
            jnp.bfloat16
        )

    @pl.when(i < NB)
    def _():
        acc = jnp.dot(
            a_ref[:].astype(jnp.bfloat16),
            h0_ref[:],
            preferred_element_type=jnp.float32,
        )
        h1_ref[pl.ds(i * BM, BM), :] = _elu(acc).astype(jnp.bfloat16)

    @pl.when(i >= NB)
    def _():
        acc = jnp.dot(
            a_ref[:].astype(jnp.bfloat16),
            h1_ref[:],
            preferred_element_type=jnp.float32,
        )
        t = _elu(acc)
        o_ref[:] = _bf16_dot(t, wo_ref[:]) + bo_ref[:]


@jax.jit
def kernel(x, adjs, W1, b1, Wout, bout):
    return pl.pallas_call(
        _fused_kernel,
        grid=(2 * NB,),
        in_specs=[
            pl.BlockSpec((BM, N), lambda i: (jax.lax.rem(i, NB), 0)),
            pl.BlockSpec((N, F), _CONST),
            pl.BlockSpec((F, F), _CONST),
            pl.BlockSpec((1, F), _CONST),
            pl.BlockSpec((F, F), _CONST),
            pl.BlockSpec((1, F), _CONST),
        ],
        out_specs=pl.BlockSpec((BM, F), lambda i: (jnp.maximum(i - NB, 0), 0)),
        out_shape=jax.ShapeDtypeStruct((N, F), jnp.float32),
        scratch_shapes=[
            pltpu.VMEM((N, F), jnp.bfloat16),
            pltpu.VMEM((N, F), jnp.bfloat16),
        ],
        compiler_params=pltpu.CompilerParams(
            vmem_limit_bytes=128 * 1024 * 1024,
        ),
    )(adjs, x, W1.T, b1.reshape(1, F), Wout.T, bout.reshape(1, F))
